# Initial kernel scaffold; baseline (speedup 1.0000x reference)
import jax
import jax.numpy as jnp
from jax.experimental import pallas as pl


def kernel(x, T, indices):
    raise NotImplementedError("write your pallas kernel here")


# speedup vs baseline: 3.2869x; 3.2869x over previous
import jax
import jax.numpy as jnp
from jax import lax
from jax.experimental import pallas as pl
from jax.experimental.pallas import tpu as pltpu

_BT = 512


def _permute_matmul_kernel(idx_ref, x_ref, t_ref, o_ref):
    o_ref[...] = lax.dot_general(
        x_ref[...],
        t_ref[0],
        dimension_numbers=(((1,), (1,)), ((), ())),
        preferred_element_type=jnp.float32,
    )


def kernel(x, T, indices):
    P, D, _ = T.shape
    B = x.shape[0]
    nb = B // _BT

    grid_spec = pltpu.PrefetchScalarGridSpec(
        num_scalar_prefetch=1,
        grid=(P, nb),
        in_specs=[
            pl.BlockSpec((_BT, D), lambda g, b, idx: (b, 0)),
            pl.BlockSpec((1, D, D), lambda g, b, idx: (idx[g], 0, 0)),
        ],
        out_specs=pl.BlockSpec((_BT, D), lambda g, b, idx: (g * nb + b, 0)),
    )
    return pl.pallas_call(
        _permute_matmul_kernel,
        out_shape=jax.ShapeDtypeStruct((P * B, D), jnp.float32),
        grid_spec=grid_spec,
        compiler_params=pltpu.CompilerParams(
            dimension_semantics=("parallel", "arbitrary"),
        ),
        name="permute_matmul",
    )(indices, x, T)


# speedup vs baseline: 4.5584x; 1.3868x over previous
import jax
import jax.numpy as jnp
from jax import lax
from jax.experimental import pallas as pl
from jax.experimental.pallas import tpu as pltpu

_BT = 1024


def _permute_matmul_kernel(idx_ref, x_ref, t_ref, o_ref):
    b = pl.program_id(1)
    o_ref[...] = lax.dot_general(
        x_ref[pl.ds(b * _BT, _BT), :],
        t_ref[0],
        dimension_numbers=(((1,), (1,)), ((), ())),
        preferred_element_type=jnp.float32,
    )


def kernel(x, T, indices):
    P, D, _ = T.shape
    B = x.shape[0]
    nb = B // _BT

    grid_spec = pltpu.PrefetchScalarGridSpec(
        num_scalar_prefetch=1,
        grid=(P, nb),
        in_specs=[
            pl.BlockSpec((B, D), lambda g, b, idx: (0, 0)),
            pl.BlockSpec((1, D, D), lambda g, b, idx: (idx[g], 0, 0)),
        ],
        out_specs=pl.BlockSpec((_BT, D), lambda g, b, idx: (g * nb + b, 0)),
    )
    return pl.pallas_call(
        _permute_matmul_kernel,
        out_shape=jax.ShapeDtypeStruct((P * B, D), jnp.float32),
        grid_spec=grid_spec,
        compiler_params=pltpu.CompilerParams(
            dimension_semantics=("parallel", "arbitrary"),
            vmem_limit_bytes=56 * 1024 * 1024,
        ),
        name="permute_matmul",
    )(indices, x, T)


# speedup vs baseline: 5.5913x; 1.2266x over previous
import jax
import jax.numpy as jnp
from jax import lax
from jax.experimental import pallas as pl
from jax.experimental.pallas import tpu as pltpu

_BT = 2048


def _permute_matmul_kernel(idx_ref, x_ref, t_ref, o_ref):
    b = pl.program_id(1)
    row = pl.multiple_of(b * _BT, _BT)
    o_ref[...] = lax.dot_general(
        x_ref[pl.ds(row, _BT), :],
        t_ref[0],
        dimension_numbers=(((1,), (1,)), ((), ())),
        preferred_element_type=jnp.float32,
    )


def kernel(x, T, indices):
    P, D, _ = T.shape
    B = x.shape[0]
    nb = B // _BT

    grid_spec = pltpu.PrefetchScalarGridSpec(
        num_scalar_prefetch=1,
        grid=(P, nb),
        in_specs=[
            pl.BlockSpec((B, D), lambda g, b, idx: (0, 0)),
            pl.BlockSpec((1, D, D), lambda g, b, idx: (idx[g], 0, 0)),
        ],
        out_specs=pl.BlockSpec((_BT, D), lambda g, b, idx: (g * nb + b, 0)),
    )
    return pl.pallas_call(
        _permute_matmul_kernel,
        out_shape=jax.ShapeDtypeStruct((P * B, D), jnp.float32),
        grid_spec=grid_spec,
        compiler_params=pltpu.CompilerParams(
            dimension_semantics=("parallel", "arbitrary"),
            vmem_limit_bytes=56 * 1024 * 1024,
        ),
        name="permute_matmul",
    )(indices, x, T)
